# baseline (speedup 1.0000x reference)
import functools

import jax
import jax.numpy as jnp
from jax import lax
from jax.experimental import pallas as pl
from jax.experimental.pallas import tpu as pltpu
from jax.experimental.pallas import tpu_sc as plsc

NX = 432
NY = 496
GRID = NX * NY
B = 4
P_PER = 10000
C_FEAT = 64
C_ALL = C_FEAT + 3
P_PAD = P_PER + 16
NSHARD = 8
SHARD = GRID // NSHARD
SENT = P_PER
KEY_SHIFT = 14

_mesh = plsc.VectorSubcoreMesh(core_axis_name="c", subcore_axis_name="s")


@functools.partial(
    pl.kernel,
    mesh=_mesh,
    out_type=[
        jax.ShapeDtypeStruct((B * C_FEAT * GRID,), jnp.float32),
        jax.ShapeDtypeStruct((B * 3 * GRID,), jnp.float32),
    ],
    scratch_types=[
        pltpu.VMEM((P_PAD,), jnp.int32),
        pltpu.VMEM((SHARD,), jnp.int32),
        pltpu.VMEM((P_PAD,), jnp.float32),
        pltpu.VMEM((SHARD,), jnp.float32),
        pltpu.VMEM((32,), jnp.int32),
    ],
    compiler_params=pltpu.CompilerParams(needs_layout_passes=False),
)
def _sc_scatter(table_hbm, keys_hbm, feat_hbm, coord_hbm,
                kbuf, wmap, tbuf, obuf, tmp32):
    cid = lax.axis_index("c")
    sid = lax.axis_index("s")
    wid = sid * 2 + cid
    b = wid // NSHARD
    sh = wid % NSHARD
    lo = sh * SHARD

    pltpu.sync_copy(keys_hbm.at[pl.ds(b * P_PAD, P_PAD)], kbuf)

    lanes = lax.broadcasted_iota(jnp.int32, (16,), 0)

    def init_body(i, carry):
        wmap[pl.ds(i * 16, 16)] = jnp.full((16,), SENT, jnp.int32)
        return carry

    lax.fori_loop(0, SHARD // 16, init_body, 0)

    tmp32[pl.ds(16, 16)] = jnp.full((16,), -1, jnp.int32)

    def p1_body(i, carry):
        idx = kbuf[pl.ds(i * 16, 16)]
        tmp32[pl.ds(0, 16)] = idx
        dup = idx < 0
        for s in range(1, 16):
            g = plsc.load_gather(tmp32, [lanes + s])
            dup = jnp.logical_or(dup, idx == g)
        memb = jnp.logical_and(idx >= lo, idx < lo + SHARD)
        mask = jnp.logical_and(jnp.logical_not(dup), memb)
        li = jnp.clip(idx - lo, 0, SHARD - 1)
        q = i * 16 + lanes
        plsc.store_scatter(wmap, [li], q, mask=mask)
        return carry

    lax.fori_loop(0, P_PAD // 16, p1_body, 0)

    def gather_body(i, carry):
        wv = wmap[pl.ds(i * 16, 16)]
        obuf[pl.ds(i * 16, 16)] = plsc.load_gather(tbuf, [wv])
        return carry

    for c in range(C_ALL):
        pltpu.sync_copy(table_hbm.at[pl.ds((b * C_ALL + c) * P_PAD, P_PAD)], tbuf)
        lax.fori_loop(0, SHARD // 16, gather_body, 0)
        if c < C_FEAT:
            dst = feat_hbm.at[pl.ds((b * C_FEAT + c) * GRID + lo, SHARD)]
        else:
            dst = coord_hbm.at[pl.ds((b * 3 + (c - C_FEAT)) * GRID + lo, SHARD)]
        pltpu.sync_copy(obuf, dst)


def kernel(pillar_features, voxel_coords):
    pfb = pillar_features.reshape(B, P_PER, C_FEAT)
    vcb = voxel_coords.reshape(B, P_PER, 4)
    z = vcb[..., 1]
    y = vcb[..., 2]
    x = vcb[..., 3]

    ftb = jnp.swapaxes(pfb, 1, 2)
    coord = jnp.stack([y, x, z], axis=1).astype(jnp.float32)
    table = jnp.concatenate([ftb, coord], axis=1)
    table = jnp.pad(table, ((0, 0), (0, 0), (0, P_PAD - P_PER)))

    idx = z + y * NX + x
    keys = jnp.pad(idx, ((0, 0), (0, P_PAD - P_PER)),
                   constant_values=jnp.int32(GRID))

    feat, coord_out = _sc_scatter(table.reshape(-1), keys.reshape(-1))
    return (feat.reshape(B, C_FEAT, NY, NX),
            coord_out.reshape(B, 3, NY, NX))


# speedup vs baseline: 1.4793x; 1.4793x over previous
import functools

import jax
import jax.numpy as jnp
from jax import lax
from jax.experimental import pallas as pl
from jax.experimental.pallas import tpu as pltpu
from jax.experimental.pallas import tpu_sc as plsc

NX = 432
NY = 496
GRID = NX * NY
B = 4
P_PER = 10000
C_FEAT = 64
C_ALL = C_FEAT + 3
P_PAD = P_PER + 16
NSHARD = 8
SHARD = GRID // NSHARD
SENT = P_PER

_mesh = plsc.VectorSubcoreMesh(core_axis_name="c", subcore_axis_name="s")


@functools.partial(
    pl.kernel,
    mesh=_mesh,
    out_type=[
        jax.ShapeDtypeStruct((B * C_FEAT * GRID,), jnp.float32),
        jax.ShapeDtypeStruct((B * 3 * GRID,), jnp.float32),
    ],
    scratch_types=[
        pltpu.VMEM((P_PAD,), jnp.int32),
        pltpu.VMEM((SHARD,), jnp.int32),
        pltpu.VMEM((P_PAD,), jnp.float32),
        pltpu.VMEM((P_PAD,), jnp.float32),
        pltpu.VMEM((SHARD,), jnp.float32),
        pltpu.VMEM((SHARD,), jnp.float32),
        pltpu.SemaphoreType.DMA,
        pltpu.SemaphoreType.DMA,
        pltpu.SemaphoreType.DMA,
        pltpu.SemaphoreType.DMA,
        pltpu.SemaphoreType.DMA,
    ],
    compiler_params=pltpu.CompilerParams(needs_layout_passes=False),
)
def _sc_scatter(table_hbm, keys_hbm, feat_hbm, coord_hbm,
                kbuf, wmap, tbuf0, tbuf1, obuf0, obuf1,
                tsem0, tsem1, osem0, osem1, ksem):
    cid = lax.axis_index("c")
    sid = lax.axis_index("s")
    wid = sid * 2 + cid
    b = wid // NSHARD
    sh = wid % NSHARD
    lo = sh * SHARD

    pltpu.async_copy(keys_hbm.at[pl.ds(b * P_PAD, P_PAD)], kbuf, ksem)

    lanes = lax.broadcasted_iota(jnp.int32, (16,), 0)

    @plsc.parallel_loop(0, SHARD // 16, unroll=8)
    def _(i):
        wmap[pl.ds(i * 16, 16)] = jnp.full((16,), SENT, jnp.int32)

    pltpu.make_async_copy(keys_hbm.at[pl.ds(b * P_PAD, P_PAD)], kbuf, ksem).wait()

    def p1_body(i, carry):
        idx = kbuf[pl.ds(i * 16, 16)]
        _, keep = plsc.scan_count(idx)
        memb = jnp.logical_and(idx >= lo, idx < lo + SHARD)
        mask = jnp.logical_and(keep, memb)
        li = jnp.clip(idx - lo, 0, SHARD - 1)
        q = i * 16 + lanes
        plsc.store_scatter(wmap, [li], q, mask=mask)
        return carry

    lax.fori_loop(0, P_PAD // 16, p1_body, 0)

    def tsrc(c):
        return table_hbm.at[pl.ds((b * C_ALL + c) * P_PAD, P_PAD)]

    def dst(c):
        if c < C_FEAT:
            return feat_hbm.at[pl.ds((b * C_FEAT + c) * GRID + lo, SHARD)]
        return coord_hbm.at[pl.ds((b * 3 + (c - C_FEAT)) * GRID + lo, SHARD)]

    tb = (tbuf0, tbuf1)
    ob = (obuf0, obuf1)
    tsem = (tsem0, tsem1)
    osem = (osem0, osem1)

    def gather_into(out_buf, table_buf):
        @plsc.parallel_loop(0, SHARD // 16, unroll=8)
        def _(i):
            wv = wmap[pl.ds(i * 16, 16)]
            out_buf[pl.ds(i * 16, 16)] = plsc.load_gather(table_buf, [wv])

    pltpu.async_copy(tsrc(0), tb[0], tsem[0])
    for c in range(C_ALL):
        cur = c & 1
        pltpu.make_async_copy(tsrc(c), tb[cur], tsem[cur]).wait()
        if c + 1 < C_ALL:
            pltpu.async_copy(tsrc(c + 1), tb[1 - cur], tsem[1 - cur])
        if c >= 2:
            pltpu.make_async_copy(ob[cur], dst(c - 2), osem[cur]).wait()
        gather_into(ob[cur], tb[cur])
        pltpu.async_copy(ob[cur], dst(c), osem[cur])
    pltpu.make_async_copy(ob[1], dst(C_ALL - 2), osem[1]).wait()
    pltpu.make_async_copy(ob[0], dst(C_ALL - 1), osem[0]).wait()


def kernel(pillar_features, voxel_coords):
    pfb = pillar_features.reshape(B, P_PER, C_FEAT)
    vcb = voxel_coords.reshape(B, P_PER, 4)
    z = vcb[..., 1]
    y = vcb[..., 2]
    x = vcb[..., 3]

    ftb = jnp.swapaxes(pfb, 1, 2)
    coord = jnp.stack([y, x, z], axis=1).astype(jnp.float32)
    table = jnp.concatenate([ftb, coord], axis=1)
    table = jnp.pad(table, ((0, 0), (0, 0), (0, P_PAD - P_PER)))

    idx = z + y * NX + x
    keys = jnp.pad(idx, ((0, 0), (0, P_PAD - P_PER)),
                   constant_values=jnp.int32(GRID))

    feat, coord_out = _sc_scatter(table.reshape(-1), keys.reshape(-1))
    return (feat.reshape(B, C_FEAT, NY, NX),
            coord_out.reshape(B, 3, NY, NX))


# speedup vs baseline: 4.4549x; 3.0116x over previous
import functools

import jax
import jax.numpy as jnp
from jax import lax
from jax.experimental import pallas as pl
from jax.experimental.pallas import tpu as pltpu
from jax.experimental.pallas import tpu_sc as plsc

NX = 432
NY = 496
GRID = NX * NY
B = 4
P_PER = 10000
C_FEAT = 64
C_ALL = C_FEAT + 3
P_PAD = P_PER + 16
ROWS = 64
BAND = ROWS * NX
SENT = P_PER

_mesh = plsc.VectorSubcoreMesh(core_axis_name="c", subcore_axis_name="s")


@functools.partial(
    pl.kernel,
    mesh=_mesh,
    out_type=[
        jax.ShapeDtypeStruct((B, C_FEAT, NY, NX), jnp.float32),
        jax.ShapeDtypeStruct((B, 3, NY, NX), jnp.float32),
    ],
    scratch_types=[
        pltpu.VMEM((P_PAD,), jnp.int32),
        pltpu.VMEM((BAND,), jnp.int32),
        pltpu.VMEM((P_PAD,), jnp.float32),
        pltpu.VMEM((P_PAD,), jnp.float32),
        pltpu.VMEM((ROWS, NX), jnp.float32),
        pltpu.VMEM((ROWS, NX), jnp.float32),
        pltpu.SemaphoreType.DMA,
        pltpu.SemaphoreType.DMA,
        pltpu.SemaphoreType.DMA,
        pltpu.SemaphoreType.DMA,
        pltpu.SemaphoreType.DMA,
    ],
    compiler_params=pltpu.CompilerParams(needs_layout_passes=False),
)
def _sc_scatter(table_hbm, keys_hbm, feat_hbm, coord_hbm,
                kbuf, wmap, tbuf0, tbuf1, obuf0, obuf1,
                tsem0, tsem1, osem0, osem1, ksem):
    cid = lax.axis_index("c")
    sid = lax.axis_index("s")
    wid = sid * 2 + cid
    b = wid // 8
    sh = wid % 8
    y0 = jnp.minimum(sh * ROWS, NY - ROWS)
    lo = y0 * NX

    pltpu.async_copy(keys_hbm.at[pl.ds(b * P_PAD, P_PAD)], kbuf, ksem)

    lanes = lax.broadcasted_iota(jnp.int32, (16,), 0)

    @plsc.parallel_loop(0, BAND // 16, unroll=8)
    def _(i):
        wmap[pl.ds(i * 16, 16)] = jnp.full((16,), SENT, jnp.int32)

    pltpu.make_async_copy(keys_hbm.at[pl.ds(b * P_PAD, P_PAD)], kbuf, ksem).wait()

    def p1_body(i, carry):
        idx = kbuf[pl.ds(i * 16, 16)]
        _, keep = plsc.scan_count(idx)
        memb = jnp.logical_and(idx >= lo, idx < lo + BAND)
        mask = jnp.logical_and(keep, memb)
        li = jnp.clip(idx - lo, 0, BAND - 1)
        q = i * 16 + lanes
        plsc.store_scatter(wmap, [li], q, mask=mask)
        return carry

    lax.fori_loop(0, P_PAD // 16, p1_body, 0)

    def tsrc(c):
        return table_hbm.at[pl.ds((b * C_ALL + c) * P_PAD, P_PAD)]

    def fdst(c):
        return feat_hbm.at[b, c, pl.ds(y0, ROWS), :]

    def cdst(j):
        return coord_hbm.at[b, j, pl.ds(y0, ROWS), :]

    def gather_into(out_buf, table_buf):
        @plsc.parallel_loop(0, ROWS, unroll=2)
        def _(r):
            for j in range(NX // 16):
                wv = wmap[pl.ds(r * NX + j * 16, 16)]
                out_buf[r, pl.ds(j * 16, 16)] = plsc.load_gather(table_buf, [wv])

    pltpu.async_copy(tsrc(0), tbuf0, tsem0)
    pltpu.async_copy(tsrc(1), tbuf1, tsem1)

    def chan_body(k, carry):
        c0 = 2 * k

        pltpu.make_async_copy(tsrc(c0), tbuf0, tsem0).wait()

        @pl.when(k > 0)
        def _():
            pltpu.make_async_copy(obuf0, fdst(c0 - 2), osem0).wait()

        gather_into(obuf0, tbuf0)
        pltpu.async_copy(obuf0, fdst(c0), osem0)
        pltpu.async_copy(tsrc(c0 + 2), tbuf0, tsem0)

        pltpu.make_async_copy(tsrc(c0 + 1), tbuf1, tsem1).wait()

        @pl.when(k > 0)
        def _():
            pltpu.make_async_copy(obuf1, fdst(c0 - 1), osem1).wait()

        gather_into(obuf1, tbuf1)
        pltpu.async_copy(obuf1, fdst(c0 + 1), osem1)
        pltpu.async_copy(tsrc(c0 + 3), tbuf1, tsem1)
        return carry

    lax.fori_loop(0, C_FEAT // 2, chan_body, 0)

    pltpu.make_async_copy(tsrc(C_FEAT), tbuf0, tsem0).wait()
    pltpu.make_async_copy(obuf0, fdst(C_FEAT - 2), osem0).wait()
    gather_into(obuf0, tbuf0)
    pltpu.async_copy(obuf0, cdst(0), osem0)
    pltpu.async_copy(tsrc(C_FEAT + 2), tbuf0, tsem0)

    pltpu.make_async_copy(tsrc(C_FEAT + 1), tbuf1, tsem1).wait()
    pltpu.make_async_copy(obuf1, fdst(C_FEAT - 1), osem1).wait()
    gather_into(obuf1, tbuf1)
    pltpu.async_copy(obuf1, cdst(1), osem1)

    pltpu.make_async_copy(tsrc(C_FEAT + 2), tbuf0, tsem0).wait()
    pltpu.make_async_copy(obuf0, cdst(0), osem0).wait()
    gather_into(obuf0, tbuf0)
    pltpu.async_copy(obuf0, cdst(2), osem0)

    pltpu.make_async_copy(obuf1, cdst(1), osem1).wait()
    pltpu.make_async_copy(obuf0, cdst(2), osem0).wait()


def kernel(pillar_features, voxel_coords):
    pfb = pillar_features.reshape(B, P_PER, C_FEAT)
    vcb = voxel_coords.reshape(B, P_PER, 4)
    z = vcb[..., 1]
    y = vcb[..., 2]
    x = vcb[..., 3]

    ftb = jnp.swapaxes(pfb, 1, 2)
    coord = jnp.stack([y, x, z], axis=1).astype(jnp.float32)
    table = jnp.concatenate([ftb, coord], axis=1)
    table = jnp.pad(table, ((0, 0), (0, 0), (0, P_PAD - P_PER)))

    idx = z + y * NX + x
    keys = jnp.pad(idx, ((0, 0), (0, P_PAD - P_PER)),
                   constant_values=jnp.int32(GRID))

    feat, coord_out = _sc_scatter(table.reshape(-1), keys.reshape(-1))
    return feat, coord_out


# speedup vs baseline: 7.7434x; 1.7382x over previous
import functools

import jax
import jax.numpy as jnp
from jax import lax
from jax.experimental import pallas as pl
from jax.experimental.pallas import tpu as pltpu
from jax.experimental.pallas import tpu_sc as plsc

NX = 432
NY = 496
B = 4
P_PER = 10000
C_FEAT = 64
C_ALL = C_FEAT + 3
P_PAD = P_PER + 16
COLS = 56
BAND = COLS * NY
SENT = P_PER

_mesh = plsc.VectorSubcoreMesh(core_axis_name="c", subcore_axis_name="s")


@functools.partial(
    pl.kernel,
    mesh=_mesh,
    out_type=[
        jax.ShapeDtypeStruct((B, C_FEAT, NX, NY), jnp.float32),
        jax.ShapeDtypeStruct((B, 3, NX, NY), jnp.float32),
    ],
    scratch_types=[
        pltpu.VMEM((P_PAD,), jnp.int32),
        pltpu.VMEM((BAND,), jnp.int32),
        pltpu.VMEM((P_PAD,), jnp.float32),
        pltpu.VMEM((P_PAD,), jnp.float32),
        pltpu.VMEM((COLS, NY), jnp.float32),
        pltpu.VMEM((COLS, NY), jnp.float32),
        pltpu.SemaphoreType.DMA,
        pltpu.SemaphoreType.DMA,
        pltpu.SemaphoreType.DMA,
        pltpu.SemaphoreType.DMA,
        pltpu.SemaphoreType.DMA,
    ],
    compiler_params=pltpu.CompilerParams(needs_layout_passes=False),
)
def _sc_scatter(table_hbm, keys_hbm, feat_hbm, coord_hbm,
                kbuf, wmap, tbuf0, tbuf1, obuf0, obuf1,
                tsem0, tsem1, osem0, osem1, ksem):
    cid = lax.axis_index("c")
    sid = lax.axis_index("s")
    wid = sid * 2 + cid
    b = wid // 8
    sh = wid % 8
    x0 = jnp.minimum(sh * COLS, NX - COLS)
    lo = x0 * NY

    pltpu.async_copy(keys_hbm.at[pl.ds(b * P_PAD, P_PAD)], kbuf, ksem)

    lanes = lax.broadcasted_iota(jnp.int32, (16,), 0)

    @plsc.parallel_loop(0, BAND // 16, unroll=8)
    def _(i):
        wmap[pl.ds(i * 16, 16)] = jnp.full((16,), SENT, jnp.int32)

    pltpu.make_async_copy(keys_hbm.at[pl.ds(b * P_PAD, P_PAD)], kbuf, ksem).wait()

    def p1_body(i, carry):
        idx = kbuf[pl.ds(i * 16, 16)]
        _, keep = plsc.scan_count(idx)
        memb = jnp.logical_and(idx >= lo, idx < lo + BAND)
        mask = jnp.logical_and(keep, memb)
        li = jnp.clip(idx - lo, 0, BAND - 1)
        q = i * 16 + lanes
        plsc.store_scatter(wmap, [li], q, mask=mask)
        return carry

    lax.fori_loop(0, P_PAD // 16, p1_body, 0)

    def tsrc(c):
        return table_hbm.at[pl.ds((b * C_ALL + c) * P_PAD, P_PAD)]

    def fdst(c):
        return feat_hbm.at[b, c, pl.ds(x0, COLS), :]

    def cdst(j):
        return coord_hbm.at[b, j, pl.ds(x0, COLS), :]

    def gather_into(out_buf, table_buf):
        @plsc.parallel_loop(0, COLS, unroll=2)
        def _(r):
            for j in range(NY // 16):
                wv = wmap[pl.ds(r * NY + j * 16, 16)]
                out_buf[r, pl.ds(j * 16, 16)] = plsc.load_gather(table_buf, [wv])

    pltpu.async_copy(tsrc(0), tbuf0, tsem0)
    pltpu.async_copy(tsrc(1), tbuf1, tsem1)

    def chan_body(k, carry):
        c0 = 2 * k

        pltpu.make_async_copy(tsrc(c0), tbuf0, tsem0).wait()

        @pl.when(k > 0)
        def _():
            pltpu.make_async_copy(obuf0, fdst(c0 - 2), osem0).wait()

        gather_into(obuf0, tbuf0)
        pltpu.async_copy(obuf0, fdst(c0), osem0)
        pltpu.async_copy(tsrc(c0 + 2), tbuf0, tsem0)

        pltpu.make_async_copy(tsrc(c0 + 1), tbuf1, tsem1).wait()

        @pl.when(k > 0)
        def _():
            pltpu.make_async_copy(obuf1, fdst(c0 - 1), osem1).wait()

        gather_into(obuf1, tbuf1)
        pltpu.async_copy(obuf1, fdst(c0 + 1), osem1)
        pltpu.async_copy(tsrc(c0 + 3), tbuf1, tsem1)
        return carry

    lax.fori_loop(0, C_FEAT // 2, chan_body, 0)

    pltpu.make_async_copy(tsrc(C_FEAT), tbuf0, tsem0).wait()
    pltpu.make_async_copy(obuf0, fdst(C_FEAT - 2), osem0).wait()
    gather_into(obuf0, tbuf0)
    pltpu.async_copy(obuf0, cdst(0), osem0)
    pltpu.async_copy(tsrc(C_FEAT + 2), tbuf0, tsem0)

    pltpu.make_async_copy(tsrc(C_FEAT + 1), tbuf1, tsem1).wait()
    pltpu.make_async_copy(obuf1, fdst(C_FEAT - 1), osem1).wait()
    gather_into(obuf1, tbuf1)
    pltpu.async_copy(obuf1, cdst(1), osem1)

    pltpu.make_async_copy(tsrc(C_FEAT + 2), tbuf0, tsem0).wait()
    pltpu.make_async_copy(obuf0, cdst(0), osem0).wait()
    gather_into(obuf0, tbuf0)
    pltpu.async_copy(obuf0, cdst(2), osem0)

    pltpu.make_async_copy(obuf1, cdst(1), osem1).wait()
    pltpu.make_async_copy(obuf0, cdst(2), osem0).wait()


def kernel(pillar_features, voxel_coords):
    pfb = pillar_features.reshape(B, P_PER, C_FEAT)
    vcb = voxel_coords.reshape(B, P_PER, 4)
    z = vcb[..., 1]
    y = vcb[..., 2]
    x = vcb[..., 3]

    ftb = jnp.swapaxes(pfb, 1, 2)
    coord = jnp.stack([y, x, z], axis=1).astype(jnp.float32)
    table = jnp.concatenate([ftb, coord], axis=1)
    table = jnp.pad(table, ((0, 0), (0, 0), (0, P_PAD - P_PER)))

    idx = x * NY + y + z
    keys = jnp.pad(idx, ((0, 0), (0, P_PAD - P_PER)),
                   constant_values=jnp.int32(NX * NY))

    feat, coord_out = _sc_scatter(table.reshape(-1), keys.reshape(-1))
    return (jnp.swapaxes(feat, 2, 3), jnp.swapaxes(coord_out, 2, 3))
